# baseline (speedup 1.0000x reference)
import jax
import jax.numpy as jnp
from jax.experimental import pallas as pl
from jax.experimental.pallas import tpu as pltpu

_N = 10000
_D = 128
_H = 32
_OUT = 128
_BK = 200
_STEPS = _N // (2 * _BK)


def _gnn_kernel(xa_ref, xb_ref, adja_ref, adjb_ref, w1t_ref, b1_ref,
                gamma_ref, beta_ref, w2_ref, b2_ref, wfct_ref, bfc_ref,
                out_ref, y_ref, zt_ref):
    k = pl.program_id(0)

    def proj(x_blk):
        return jax.lax.dot_general(
            x_blk, w1t_ref[...], (((1,), (0,)), ((), ())),
            preferred_element_type=jnp.float32,
            precision=jax.lax.Precision.HIGHEST)

    ya = proj(xa_ref[...])
    yb = proj(xb_ref[...])
    y_ref[pl.ds((2 * k) * _BK, _BK), :] = ya
    y_ref[pl.ds((2 * k + 1) * _BK, _BK), :] = yb

    def msg(y_blk, adj_ref):
        return jax.lax.dot_general(
            y_blk, adj_ref[...], (((0,), (0,)), ((), ())),
            preferred_element_type=jnp.float32,
            precision=jax.lax.Precision.DEFAULT)

    zpart = msg(ya, adja_ref) + msg(yb, adjb_ref)

    @pl.when(k == 0)
    def _():
        zt_ref[...] = zpart

    @pl.when(k > 0)
    def _():
        zt_ref[...] += zpart

    @pl.when(k == _STEPS - 1)
    def _():
        yt = y_ref[...].T
        ht = yt + zt_ref[...] + b1_ref[...]
        mu = jnp.mean(ht, axis=1, keepdims=True)
        d = ht - mu
        var = jnp.mean(d * d, axis=1, keepdims=True)
        hn = d * jax.lax.rsqrt(var + 1e-5) * gamma_ref[...] + beta_ref[...]
        hr = jnp.maximum(hn, 0.0)
        h2 = jax.lax.dot_general(
            w2_ref[...], hr, (((1,), (0,)), ((), ())),
            preferred_element_type=jnp.float32,
            precision=jax.lax.Precision.HIGHEST) + b2_ref[...]
        h2 = jnp.maximum(h2, 0.0)
        pooled = jnp.mean(h2, axis=1, keepdims=True)
        out = jax.lax.dot_general(
            pooled, wfct_ref[...], (((0,), (0,)), ((), ())),
            preferred_element_type=jnp.float32,
            precision=jax.lax.Precision.HIGHEST) + bfc_ref[...]
        out_ref[...] = out


def kernel(x, adj, W1, b1, gamma, beta, W2, b2, Wfc, bfc):
    w1t = W1.T
    wfct = Wfc.T
    b1c = b1.reshape(_H, 1)
    gammac = gamma.reshape(_H, 1)
    betac = beta.reshape(_H, 1)
    b2c = b2.reshape(_H, 1)
    bfcr = bfc.reshape(1, _OUT)

    return pl.pallas_call(
        _gnn_kernel,
        grid=(_STEPS,),
        in_specs=[
            pl.BlockSpec((_BK, _D), lambda k: (2 * k, 0)),
            pl.BlockSpec((_BK, _D), lambda k: (2 * k + 1, 0)),
            pl.BlockSpec((_BK, _N), lambda k: (2 * k, 0)),
            pl.BlockSpec((_BK, _N), lambda k: (2 * k + 1, 0)),
            pl.BlockSpec((_D, _H), lambda k: (0, 0)),
            pl.BlockSpec((_H, 1), lambda k: (0, 0)),
            pl.BlockSpec((_H, 1), lambda k: (0, 0)),
            pl.BlockSpec((_H, 1), lambda k: (0, 0)),
            pl.BlockSpec((_H, _H), lambda k: (0, 0)),
            pl.BlockSpec((_H, 1), lambda k: (0, 0)),
            pl.BlockSpec((_H, _OUT), lambda k: (0, 0)),
            pl.BlockSpec((1, _OUT), lambda k: (0, 0)),
        ],
        out_specs=pl.BlockSpec((1, _OUT), lambda k: (0, 0)),
        out_shape=jax.ShapeDtypeStruct((1, _OUT), jnp.float32),
        scratch_shapes=[
            pltpu.VMEM((_N, _H), jnp.float32),
            pltpu.VMEM((_H, _N), jnp.float32),
        ],
        compiler_params=pltpu.CompilerParams(
            dimension_semantics=("arbitrary",)),
    )(x, x, adj, adj, w1t, b1c, gammac, betac, W2, b2c, wfct, bfcr)


# speedup vs baseline: 1.0008x; 1.0008x over previous
import jax
import jax.numpy as jnp
from jax.experimental import pallas as pl
from jax.experimental.pallas import tpu as pltpu

_N = 10000
_D = 128
_H = 32
_OUT = 128
_BK = 400
_STEPS = _N // _BK


def _gnn_kernel(x_ref, adj_ref, w1t_ref, b1_ref, gamma_ref, beta_ref,
                w2_ref, b2_ref, wfct_ref, bfc_ref, out_ref,
                y_ref, zt_ref):
    k = pl.program_id(0)

    xb = x_ref[...]
    adjb = adj_ref[...]

    yb = jax.lax.dot_general(
        xb, w1t_ref[...], (((1,), (0,)), ((), ())),
        preferred_element_type=jnp.float32,
        precision=jax.lax.Precision.HIGHEST)
    y_ref[pl.ds(k * _BK, _BK), :] = yb

    zpart = jax.lax.dot_general(
        yb, adjb,
        (((0,), (0,)), ((), ())),
        preferred_element_type=jnp.float32,
        precision=jax.lax.Precision.DEFAULT)

    @pl.when(k == 0)
    def _():
        zt_ref[...] = zpart

    @pl.when(k > 0)
    def _():
        zt_ref[...] += zpart

    @pl.when(k == _STEPS - 1)
    def _():
        yt = y_ref[...].T
        ht = yt + zt_ref[...] + b1_ref[...]
        mu = jnp.mean(ht, axis=1, keepdims=True)
        d = ht - mu
        var = jnp.mean(d * d, axis=1, keepdims=True)
        hn = d * jax.lax.rsqrt(var + 1e-5) * gamma_ref[...] + beta_ref[...]
        hr = jnp.maximum(hn, 0.0)
        h2 = jax.lax.dot_general(
            w2_ref[...], hr, (((1,), (0,)), ((), ())),
            preferred_element_type=jnp.float32,
            precision=jax.lax.Precision.HIGHEST) + b2_ref[...]
        h2 = jnp.maximum(h2, 0.0)
        pooled = jnp.mean(h2, axis=1, keepdims=True)
        out = jax.lax.dot_general(
            pooled, wfct_ref[...], (((0,), (0,)), ((), ())),
            preferred_element_type=jnp.float32,
            precision=jax.lax.Precision.HIGHEST) + bfc_ref[...]
        out_ref[...] = out


def kernel(x, adj, W1, b1, gamma, beta, W2, b2, Wfc, bfc):
    w1t = W1.T
    wfct = Wfc.T
    b1c = b1.reshape(_H, 1)
    gammac = gamma.reshape(_H, 1)
    betac = beta.reshape(_H, 1)
    b2c = b2.reshape(_H, 1)
    bfcr = bfc.reshape(1, _OUT)

    return pl.pallas_call(
        _gnn_kernel,
        grid=(_STEPS,),
        in_specs=[
            pl.BlockSpec((_BK, _D), lambda k: (k, 0)),
            pl.BlockSpec((_BK, _N), lambda k: (k, 0)),
            pl.BlockSpec((_D, _H), lambda k: (0, 0)),
            pl.BlockSpec((_H, 1), lambda k: (0, 0)),
            pl.BlockSpec((_H, 1), lambda k: (0, 0)),
            pl.BlockSpec((_H, 1), lambda k: (0, 0)),
            pl.BlockSpec((_H, _H), lambda k: (0, 0)),
            pl.BlockSpec((_H, 1), lambda k: (0, 0)),
            pl.BlockSpec((_H, _OUT), lambda k: (0, 0)),
            pl.BlockSpec((1, _OUT), lambda k: (0, 0)),
        ],
        out_specs=pl.BlockSpec((1, _OUT), lambda k: (0, 0)),
        out_shape=jax.ShapeDtypeStruct((1, _OUT), jnp.float32),
        scratch_shapes=[
            pltpu.VMEM((_N, _H), jnp.float32),
            pltpu.VMEM((_H, _N), jnp.float32),
        ],
        compiler_params=pltpu.CompilerParams(
            dimension_semantics=("arbitrary",)),
    )(x, adj, w1t, b1c, gammac, betac, W2, b2c, wfct, bfcr)


# speedup vs baseline: 1.0412x; 1.0404x over previous
import jax
import jax.numpy as jnp
from jax.experimental import pallas as pl
from jax.experimental.pallas import tpu as pltpu

_N = 10000
_D = 128
_H = 32
_OUT = 128
_BK = 400
_STEPS = _N // _BK


def _gnn_kernel(x_ref, adj_ref, w1_ref, b1_ref, gamma_ref, beta_ref,
                w2_ref, b2_ref, wfc_ref, bfc_ref, out_ref,
                y_ref, zt_ref):
    k = pl.program_id(0)

    xb = x_ref[...]
    adjb = adj_ref[...]

    yb = jax.lax.dot_general(
        xb, w1_ref[...], (((1,), (1,)), ((), ())),
        preferred_element_type=jnp.float32,
        precision=jax.lax.Precision.HIGHEST)
    y_ref[pl.ds(k * _BK, _BK), :] = yb

    zpart = jax.lax.dot_general(
        yb, adjb,
        (((0,), (0,)), ((), ())),
        preferred_element_type=jnp.float32,
        precision=jax.lax.Precision.DEFAULT)

    @pl.when(k == 0)
    def _():
        zt_ref[...] = zpart

    @pl.when(k > 0)
    def _():
        zt_ref[...] += zpart

    @pl.when(k == _STEPS - 1)
    def _():
        b1c = b1_ref[...][:, None]
        gammac = gamma_ref[...][:, None]
        betac = beta_ref[...][:, None]
        b2c = b2_ref[...][:, None]
        yt = y_ref[...].T
        ht = yt + zt_ref[...] + b1c
        mu = jnp.mean(ht, axis=1, keepdims=True)
        d = ht - mu
        var = jnp.mean(d * d, axis=1, keepdims=True)
        hn = d * jax.lax.rsqrt(var + 1e-5) * gammac + betac
        hr = jnp.maximum(hn, 0.0)
        h2 = jax.lax.dot_general(
            w2_ref[...], hr, (((1,), (0,)), ((), ())),
            preferred_element_type=jnp.float32,
            precision=jax.lax.Precision.HIGHEST) + b2c
        h2 = jnp.maximum(h2, 0.0)
        pooled = jnp.mean(h2, axis=1, keepdims=True)
        out = jax.lax.dot_general(
            pooled, wfc_ref[...], (((0,), (1,)), ((), ())),
            preferred_element_type=jnp.float32,
            precision=jax.lax.Precision.HIGHEST) + bfc_ref[...][None, :]
        out_ref[...] = out


def kernel(x, adj, W1, b1, gamma, beta, W2, b2, Wfc, bfc):
    return pl.pallas_call(
        _gnn_kernel,
        grid=(_STEPS,),
        in_specs=[
            pl.BlockSpec((_BK, _D), lambda k: (k, 0)),
            pl.BlockSpec((_BK, _N), lambda k: (k, 0)),
            pl.BlockSpec((_H, _D), lambda k: (0, 0)),
            pl.BlockSpec((_H,), lambda k: (0,)),
            pl.BlockSpec((_H,), lambda k: (0,)),
            pl.BlockSpec((_H,), lambda k: (0,)),
            pl.BlockSpec((_H, _H), lambda k: (0, 0)),
            pl.BlockSpec((_H,), lambda k: (0,)),
            pl.BlockSpec((_OUT, _H), lambda k: (0, 0)),
            pl.BlockSpec((_OUT,), lambda k: (0,)),
        ],
        out_specs=pl.BlockSpec((1, _OUT), lambda k: (0, 0)),
        out_shape=jax.ShapeDtypeStruct((1, _OUT), jnp.float32),
        scratch_shapes=[
            pltpu.VMEM((_N, _H), jnp.float32),
            pltpu.VMEM((_H, _N), jnp.float32),
        ],
        compiler_params=pltpu.CompilerParams(
            dimension_semantics=("arbitrary",)),
    )(x, adj, W1, b1, gamma, beta, W2, b2, Wfc, bfc)


# speedup vs baseline: 1.0460x; 1.0046x over previous
import jax
import jax.numpy as jnp
from jax.experimental import pallas as pl
from jax.experimental.pallas import tpu as pltpu

_N = 10000
_D = 128
_H = 32
_OUT = 128
_BK = 400
_STEPS = _N // _BK


def _gnn_kernel(x_ref, adj_ref, w1_ref, b1_ref, gamma_ref, beta_ref,
                w2_ref, b2_ref, wfc_ref, bfc_ref, out_ref,
                y_ref, zt_ref):
    k = pl.program_id(0)

    xb = x_ref[...]
    adjb = adj_ref[...]

    yb = jax.lax.dot_general(
        xb, w1_ref[...], (((1,), (1,)), ((), ())),
        preferred_element_type=jnp.float32,
        precision=jax.lax.Precision.DEFAULT)
    y_ref[pl.ds(k * _BK, _BK), :] = yb

    zpart = jax.lax.dot_general(
        yb, adjb,
        (((0,), (0,)), ((), ())),
        preferred_element_type=jnp.float32,
        precision=jax.lax.Precision.DEFAULT)

    @pl.when(k == 0)
    def _():
        zt_ref[...] = zpart

    @pl.when(k > 0)
    def _():
        zt_ref[...] += zpart

    @pl.when(k == _STEPS - 1)
    def _():
        b1c = b1_ref[...][:, None]
        gammac = gamma_ref[...][:, None]
        betac = beta_ref[...][:, None]
        b2c = b2_ref[...][:, None]
        yt = y_ref[...].T
        ht = yt + zt_ref[...] + b1c
        mu = jnp.mean(ht, axis=1, keepdims=True)
        d = ht - mu
        var = jnp.mean(d * d, axis=1, keepdims=True)
        hn = d * jax.lax.rsqrt(var + 1e-5) * gammac + betac
        hr = jnp.maximum(hn, 0.0)
        h2 = jax.lax.dot_general(
            w2_ref[...], hr, (((1,), (0,)), ((), ())),
            preferred_element_type=jnp.float32,
            precision=jax.lax.Precision.HIGHEST) + b2c
        h2 = jnp.maximum(h2, 0.0)
        pooled = jnp.mean(h2, axis=1, keepdims=True)
        out = jax.lax.dot_general(
            pooled, wfc_ref[...], (((0,), (1,)), ((), ())),
            preferred_element_type=jnp.float32,
            precision=jax.lax.Precision.HIGHEST) + bfc_ref[...][None, :]
        out_ref[...] = out


def kernel(x, adj, W1, b1, gamma, beta, W2, b2, Wfc, bfc):
    return pl.pallas_call(
        _gnn_kernel,
        grid=(_STEPS,),
        in_specs=[
            pl.BlockSpec((_BK, _D), lambda k: (k, 0)),
            pl.BlockSpec((_BK, _N), lambda k: (k, 0)),
            pl.BlockSpec((_H, _D), lambda k: (0, 0)),
            pl.BlockSpec((_H,), lambda k: (0,)),
            pl.BlockSpec((_H,), lambda k: (0,)),
            pl.BlockSpec((_H,), lambda k: (0,)),
            pl.BlockSpec((_H, _H), lambda k: (0, 0)),
            pl.BlockSpec((_H,), lambda k: (0,)),
            pl.BlockSpec((_OUT, _H), lambda k: (0, 0)),
            pl.BlockSpec((_OUT,), lambda k: (0,)),
        ],
        out_specs=pl.BlockSpec((1, _OUT), lambda k: (0, 0)),
        out_shape=jax.ShapeDtypeStruct((1, _OUT), jnp.float32),
        scratch_shapes=[
            pltpu.VMEM((_N, _H), jnp.float32),
            pltpu.VMEM((_H, _N), jnp.float32),
        ],
        compiler_params=pltpu.CompilerParams(
            dimension_semantics=("arbitrary",)),
    )(x, adj, W1, b1, gamma, beta, W2, b2, Wfc, bfc)
